# Initial kernel scaffold; baseline (speedup 1.0000x reference)
import jax
import jax.numpy as jnp
from jax.experimental import pallas as pl


def kernel(missing_mask, mode_id, missing_table, mode_table, W, b):
    raise NotImplementedError("write your pallas kernel here")


# speedup vs baseline: 2.0201x; 2.0201x over previous
import functools

import jax
import jax.numpy as jnp
from jax import lax
from jax.experimental import pallas as pl
from jax.experimental.pallas import tpu as pltpu
from jax.experimental.pallas import tpu_sc as plsc

EMBED_DIM = 64
HALF = EMBED_DIM // 2
BATCH = 16384
MODE_PAD = 8
TABLE_ROWS = 16 * MODE_PAD

NC = 2
NS = 16
L = 16
NW = NC * NS
BPW = BATCH // NW
GCH = 128
NG = BPW // GCH


def _table_body(miss_ref, mode_ref, w1t_ref, w2t_ref, b_ref, out_ref):
    miss_proj = jnp.dot(miss_ref[...], w1t_ref[...],
                        preferred_element_type=jnp.float32)
    mode_proj = jnp.dot(mode_ref[...], w2t_ref[...],
                        preferred_element_type=jnp.float32)
    out_ref[...] = (miss_proj[:, None, :] + mode_proj[None, :, :]
                    + b_ref[...][None])


def _build_table(missing_table, mode_table, W, b):
    w1t = W[:, :HALF].T
    w2t = W[:, HALF:].T
    mode_pad = jnp.zeros((MODE_PAD, HALF), jnp.float32).at[:5].set(mode_table)
    t3 = pl.pallas_call(
        _table_body,
        out_shape=jax.ShapeDtypeStruct((16, MODE_PAD, EMBED_DIM), jnp.float32),
    )(missing_table, mode_pad, w1t, w2t, b.reshape(1, EMBED_DIM))
    return t3.reshape(TABLE_ROWS, EMBED_DIM)


@functools.cache
def _make_sc_gather():
    mesh = plsc.VectorSubcoreMesh(core_axis_name="c", subcore_axis_name="s")

    @functools.partial(
        pl.kernel,
        mesh=mesh,
        compiler_params=pltpu.CompilerParams(use_tc_tiling_on_sc=False),
        out_type=jax.ShapeDtypeStruct((BATCH, EMBED_DIM), jnp.float32),
        scratch_types=[
            pltpu.VMEM((4, BPW), jnp.int32),
            pltpu.VMEM((BPW,), jnp.int32),
            pltpu.VMEM((NG, GCH), jnp.int32),
            pltpu.VMEM((BPW, EMBED_DIM), jnp.float32),
            pltpu.SemaphoreType.DMA,
        ],
    )
    def _sc_gather(maskT_hbm, mode_hbm, table_hbm, out_hbm,
                   mask_v, mode_v, idx_v, rows_v, sem):
        wid = lax.axis_index("s") * NC + lax.axis_index("c")
        base = wid * BPW

        for j in range(4):
            pltpu.sync_copy(maskT_hbm.at[j, pl.ds(base, BPW)], mask_v.at[j])
        pltpu.sync_copy(mode_hbm.at[pl.ds(base, BPW)], mode_v)

        copies = []
        for g in range(NG):
            gbase = g * GCH
            for i in range(GCH // L):
                off = gbase + i * L
                m0 = mask_v[0, pl.ds(off, L)]
                m1 = mask_v[1, pl.ds(off, L)]
                m2 = mask_v[2, pl.ds(off, L)]
                m3 = mask_v[3, pl.ds(off, L)]
                md = mode_v[pl.ds(off, L)]
                idx_v[g, pl.ds(i * L, L)] = (
                    m0 * 64 + m1 * 32 + m2 * 16 + m3 * 8 + md)
            copies.append(pltpu.async_copy(
                table_hbm.at[idx_v.at[g]], rows_v.at[pl.ds(gbase, GCH)], sem))
        for c in copies:
            c.wait()
        pltpu.sync_copy(rows_v, out_hbm.at[pl.ds(base, BPW)])

    return _sc_gather


def kernel(missing_mask, mode_id, missing_table, mode_table, W, b):
    table = _build_table(missing_table, mode_table, W, b)
    maskT = missing_mask.astype(jnp.int32).T
    mode32 = mode_id.astype(jnp.int32)
    return _make_sc_gather()(maskT, mode32, table)
